# baseline (speedup 1.0000x reference)
import functools

import jax
import jax.numpy as jnp
from jax import lax
from jax.experimental import pallas as pl
from jax.experimental.pallas import tpu as pltpu
from jax.experimental.pallas import tpu_sc as plsc

N = 10000
D = 128
E = 320000

NC = 2
NS = 16
NW = NC * NS
CHUNK = 128
EPT = 10240
E_PAD = NW * EPT
NITER2 = EPT // CHUNK
NITER1 = (E_PAD // NS) // CHUNK
IB = 40
NPAD = 10016
ZROWS = 640

_mesh = plsc.VectorSubcoreMesh(core_axis_name="c", subcore_axis_name="s")


def _tile_row_base(s):
    return jnp.minimum(s * ZROWS, N - ZROWS)


@functools.partial(
    pl.kernel,
    out_type=jax.ShapeDtypeStruct((NC, N, D), jnp.float32),
    mesh=_mesh,
    scratch_types=[
        pltpu.VMEM((NITER2, CHUNK), jnp.int32),
        pltpu.VMEM((CHUNK, D), jnp.float32),
        pltpu.VMEM_SHARED((NPAD, D), jnp.float32),
    ],
)
def _sc_degree(dst_hbm, ones_hbm, zero_hbm, cnt_hbm, idx_v, ones_v, acc_sh):
    c = lax.axis_index("c")
    s = lax.axis_index("s")
    wid = c * NS + s
    base = _tile_row_base(s)
    pltpu.sync_copy(zero_hbm, acc_sh.at[pl.ds(base, ZROWS)])
    pltpu.sync_copy(ones_hbm, ones_v)
    pltpu.sync_copy(dst_hbm.at[wid], idx_v)
    plsc.subcore_barrier()

    def body(j, carry):
        pltpu.sync_copy(ones_v, acc_sh.at[idx_v.at[j]], add=True)
        return carry

    lax.fori_loop(0, NITER2, body, 0)
    plsc.subcore_barrier()
    pltpu.sync_copy(acc_sh.at[pl.ds(base, ZROWS)],
                    cnt_hbm.at[c, pl.ds(base, ZROWS)])


def _edge_agg_body(niter, table_hbm, src_hbm, dst_hbm, zero_hbm, out_hbm,
                   src_v, dst_v, rows0_v, rows1_v, sem0, sem1, acc_sh,
                   split_by_core):
    c = lax.axis_index("c")
    s = lax.axis_index("s")
    base = _tile_row_base(s)
    pltpu.sync_copy(zero_hbm, acc_sh.at[pl.ds(base, ZROWS)])
    plsc.subcore_barrier()

    def outer(b, carry):
        if split_by_core:
            pltpu.sync_copy(src_hbm.at[c * NS + s, pl.ds(b * IB, IB)], src_v)
            pltpu.sync_copy(dst_hbm.at[c * NS + s, pl.ds(b * IB, IB)], dst_v)
        else:
            pltpu.sync_copy(src_hbm.at[c, s, pl.ds(b * IB, IB)], src_v)
            pltpu.sync_copy(dst_hbm.at[s, pl.ds(b * IB, IB)], dst_v)

        pltpu.async_copy(table_hbm.at[src_v.at[0]], rows0_v, sem0)

        def body(i, carry2):
            j0 = i * 2
            pltpu.async_copy(table_hbm.at[src_v.at[j0 + 1]], rows1_v, sem1)
            pltpu.make_async_copy(table_hbm.at[src_v.at[j0]], rows0_v,
                                  sem0).wait()
            pltpu.sync_copy(rows0_v, acc_sh.at[dst_v.at[j0]], add=True)

            @pl.when(i + 1 < IB // 2)
            def _():
                pltpu.async_copy(table_hbm.at[src_v.at[j0 + 2]], rows0_v,
                                 sem0)

            pltpu.make_async_copy(table_hbm.at[src_v.at[j0 + 1]], rows1_v,
                                  sem1).wait()
            pltpu.sync_copy(rows1_v, acc_sh.at[dst_v.at[j0 + 1]], add=True)
            return carry2

        lax.fori_loop(0, IB // 2, body, 0)
        return carry

    lax.fori_loop(0, niter // IB, outer, 0)
    plsc.subcore_barrier()
    pltpu.sync_copy(acc_sh.at[pl.ds(base, ZROWS)],
                    out_hbm.at[c, pl.ds(base, ZROWS)])


@functools.partial(
    pl.kernel,
    out_type=jax.ShapeDtypeStruct((NC, N, D), jnp.float32),
    mesh=_mesh,
    scratch_types=[
        pltpu.VMEM((IB, CHUNK), jnp.int32),
        pltpu.VMEM((IB, CHUNK), jnp.int32),
        pltpu.VMEM((CHUNK, D), jnp.float32),
        pltpu.VMEM((CHUNK, D), jnp.float32),
        pltpu.SemaphoreType.DMA,
        pltpu.SemaphoreType.DMA,
        pltpu.VMEM_SHARED((NPAD, D), jnp.float32),
    ],
)
def _sc_agg1(table_hbm, src_hbm, dst_hbm, zero_hbm, out_hbm,
             src_v, dst_v, rows0_v, rows1_v, sem0, sem1, acc_sh):
    _edge_agg_body(NITER1, table_hbm, src_hbm, dst_hbm, zero_hbm, out_hbm,
                   src_v, dst_v, rows0_v, rows1_v, sem0, sem1, acc_sh,
                   split_by_core=False)


@functools.partial(
    pl.kernel,
    out_type=jax.ShapeDtypeStruct((NC, N, D), jnp.float32),
    mesh=_mesh,
    scratch_types=[
        pltpu.VMEM((IB, CHUNK), jnp.int32),
        pltpu.VMEM((IB, CHUNK), jnp.int32),
        pltpu.VMEM((CHUNK, D), jnp.float32),
        pltpu.VMEM((CHUNK, D), jnp.float32),
        pltpu.SemaphoreType.DMA,
        pltpu.SemaphoreType.DMA,
        pltpu.VMEM_SHARED((NPAD, D), jnp.float32),
    ],
)
def _sc_agg2(table_hbm, src_hbm, dst_hbm, zero_hbm, out_hbm,
             src_v, dst_v, rows0_v, rows1_v, sem0, sem1, acc_sh):
    _edge_agg_body(NITER2, table_hbm, src_hbm, dst_hbm, zero_hbm, out_hbm,
                   src_v, dst_v, rows0_v, rows1_v, sem0, sem1, acc_sh,
                   split_by_core=True)


BN = 2000


def _tc1_body(emb_ref, w1_ref, cnt_ref, xs_ref, dinv_ref):
    x1 = jnp.dot(emb_ref[...], w1_ref[...], preferred_element_type=jnp.float32)
    deg = 1.0 + cnt_ref[0, :, 0:1] + cnt_ref[1, :, 0:1]
    dinv = lax.rsqrt(deg)
    dinv_ref[...] = dinv
    xs = x1 * dinv
    xs_ref[0] = xs[:, :D]
    xs_ref[1] = xs[:, D:]


def _tc1(emb, W1, cnt):
    return pl.pallas_call(
        _tc1_body,
        grid=(N // BN,),
        in_specs=[
            pl.BlockSpec((BN, D), lambda i: (i, 0)),
            pl.BlockSpec((D, 2 * D), lambda i: (0, 0)),
            pl.BlockSpec((NC, BN, D), lambda i: (0, i, 0)),
        ],
        out_specs=[
            pl.BlockSpec((NC, BN, D), lambda i: (0, i, 0)),
            pl.BlockSpec((BN, 1), lambda i: (i, 0)),
        ],
        out_shape=[
            jax.ShapeDtypeStruct((NC, N, D), jnp.float32),
            jax.ShapeDtypeStruct((N, 1), jnp.float32),
        ],
    )(emb, W1, cnt)


def _tc2_body(agg_ref, xs_ref, dinv_ref, b1_ref, w2_ref, xs2_ref):
    dinv = dinv_ref[...]
    h = jnp.concatenate(
        [agg_ref[0] + xs_ref[0], agg_ref[1] + xs_ref[1]], axis=1)
    h = h * dinv + b1_ref[...]
    nrm = jnp.sqrt(jnp.sum(h * h, axis=1, keepdims=True))
    h = h / jnp.maximum(nrm, 1e-12)
    x2 = jnp.dot(h, w2_ref[...], preferred_element_type=jnp.float32)
    xs2_ref[...] = x2 * dinv


def _tc2(agg, xs, dinv, b1, W2):
    return pl.pallas_call(
        _tc2_body,
        grid=(N // BN,),
        in_specs=[
            pl.BlockSpec((NC, BN, D), lambda i: (0, i, 0)),
            pl.BlockSpec((NC, BN, D), lambda i: (0, i, 0)),
            pl.BlockSpec((BN, 1), lambda i: (i, 0)),
            pl.BlockSpec((1, 2 * D), lambda i: (0, 0)),
            pl.BlockSpec((2 * D, D), lambda i: (0, 0)),
        ],
        out_specs=pl.BlockSpec((BN, D), lambda i: (i, 0)),
        out_shape=jax.ShapeDtypeStruct((N, D), jnp.float32),
    )(agg, xs, dinv, b1, W2)


def _tc3_body(p_ref, xs2_ref, dinv_ref, b2_ref, out_ref):
    out_ref[...] = (dinv_ref[...] * (p_ref[0] + p_ref[1] + xs2_ref[...])
                    + b2_ref[...])


def _tc3(p, xs2, dinv, b2):
    return pl.pallas_call(
        _tc3_body,
        grid=(N // BN,),
        in_specs=[
            pl.BlockSpec((NC, BN, D), lambda i: (0, i, 0)),
            pl.BlockSpec((BN, D), lambda i: (i, 0)),
            pl.BlockSpec((BN, 1), lambda i: (i, 0)),
            pl.BlockSpec((1, D), lambda i: (0, 0)),
        ],
        out_specs=pl.BlockSpec((BN, D), lambda i: (i, 0)),
        out_shape=jax.ShapeDtypeStruct((N, D), jnp.float32),
    )(p, xs2, dinv, b2)


def kernel(edge_index, emb, W1, b1, W2, b2):
    src = edge_index[0].astype(jnp.int32)
    dst = edge_index[1].astype(jnp.int32)
    pad = E_PAD - E
    src_p = jnp.concatenate([src, jnp.zeros((pad,), jnp.int32)])
    dst_p = jnp.concatenate([dst, jnp.full((pad,), N, jnp.int32)])

    src32 = src_p.reshape(NW, NITER2, CHUNK)
    dst32 = dst_p.reshape(NW, NITER2, CHUNK)
    src16 = src_p.reshape(NS, NITER1, CHUNK)
    src16_2 = jnp.stack([src16, src16 + N])
    dst16 = dst_p.reshape(NS, NITER1, CHUNK)

    ones128 = jnp.ones((CHUNK, D), jnp.float32)
    zeros128 = jnp.zeros((ZROWS, D), jnp.float32)

    cnt = _sc_degree(dst32, ones128, zeros128)
    xs, dinv = _tc1(emb, W1, cnt)
    agg = _sc_agg1(xs.reshape(NC * N, D), src16_2, dst16, zeros128)
    xs2 = _tc2(agg, xs, dinv, b1.reshape(1, 2 * D), W2)
    p = _sc_agg2(xs2, src32, dst32, zeros128)
    return _tc3(p, xs2, dinv, b2.reshape(1, D))


# speedup vs baseline: 2.8024x; 2.8024x over previous
import functools

import jax
import jax.numpy as jnp
from jax import lax
from jax.experimental import pallas as pl
from jax.experimental.pallas import tpu as pltpu
from jax.experimental.pallas import tpu_sc as plsc

N = 10000
D = 128
E = 320000

NC = 2
NS = 16
NW = NC * NS
CHUNK = 128
EPT = 10240
E_PAD = NW * EPT
NITER2 = EPT // CHUNK
NITER1 = (E_PAD // NS) // CHUNK
IB = 40
NPAD = 10016
ZROWS = 640

_mesh = plsc.VectorSubcoreMesh(core_axis_name="c", subcore_axis_name="s")


def _tile_row_base(s):
    return jnp.minimum(s * ZROWS, N - ZROWS)


@functools.partial(
    pl.kernel,
    out_type=jax.ShapeDtypeStruct((NC, N, D), jnp.float32),
    mesh=_mesh,
    scratch_types=[
        pltpu.VMEM((NITER2, CHUNK), jnp.int32),
        pltpu.VMEM((CHUNK, D), jnp.float32),
        pltpu.VMEM_SHARED((NPAD, D), jnp.float32),
    ],
)
def _sc_degree(dst_hbm, ones_hbm, zero_hbm, cnt_hbm, idx_v, ones_v, acc_sh):
    c = lax.axis_index("c")
    s = lax.axis_index("s")
    wid = c * NS + s
    base = _tile_row_base(s)
    pltpu.sync_copy(zero_hbm, acc_sh.at[pl.ds(base, ZROWS)])
    pltpu.sync_copy(ones_hbm, ones_v)
    pltpu.sync_copy(dst_hbm.at[wid], idx_v)
    plsc.subcore_barrier()

    def body(j, carry):
        pltpu.sync_copy(ones_v, acc_sh.at[idx_v.at[j]], add=True)
        return carry

    lax.fori_loop(0, NITER2, body, 0)
    plsc.subcore_barrier()
    pltpu.sync_copy(acc_sh.at[pl.ds(base, ZROWS)],
                    cnt_hbm.at[c, pl.ds(base, ZROWS)])


def _edge_agg_body(niter, table_hbm, src_hbm, dst_hbm, zero_hbm, out_hbm,
                   src_v, dst_v, rows0_v, rows1_v, sem0, sem1, acc_sh,
                   split_by_core):
    c = lax.axis_index("c")
    s = lax.axis_index("s")
    base = _tile_row_base(s)
    pltpu.sync_copy(zero_hbm, acc_sh.at[pl.ds(base, ZROWS)])
    plsc.subcore_barrier()

    def outer(b, carry):
        if split_by_core:
            pltpu.sync_copy(src_hbm.at[c * NS + s, pl.ds(b * IB, IB)], src_v)
            pltpu.sync_copy(dst_hbm.at[c * NS + s, pl.ds(b * IB, IB)], dst_v)
        else:
            pltpu.sync_copy(src_hbm.at[c, s, pl.ds(b * IB, IB)], src_v)
            pltpu.sync_copy(dst_hbm.at[s, pl.ds(b * IB, IB)], dst_v)

        pltpu.async_copy(table_hbm.at[src_v.at[0]], rows0_v, sem0)

        def body(i, carry2):
            j0 = i * 2
            pltpu.async_copy(table_hbm.at[src_v.at[j0 + 1]], rows1_v, sem1)
            pltpu.make_async_copy(table_hbm.at[src_v.at[j0]], rows0_v,
                                  sem0).wait()
            pltpu.sync_copy(rows0_v, acc_sh.at[dst_v.at[j0]], add=True)

            @pl.when(i + 1 < IB // 2)
            def _():
                pltpu.async_copy(table_hbm.at[src_v.at[j0 + 2]], rows0_v,
                                 sem0)

            pltpu.make_async_copy(table_hbm.at[src_v.at[j0 + 1]], rows1_v,
                                  sem1).wait()
            pltpu.sync_copy(rows1_v, acc_sh.at[dst_v.at[j0 + 1]], add=True)
            return carry2

        lax.fori_loop(0, IB // 2, body, 0)
        return carry

    lax.fori_loop(0, niter // IB, outer, 0)
    plsc.subcore_barrier()
    pltpu.sync_copy(acc_sh.at[pl.ds(base, ZROWS)],
                    out_hbm.at[c, pl.ds(base, ZROWS)])


@functools.partial(
    pl.kernel,
    out_type=jax.ShapeDtypeStruct((NC, N, D), jnp.float32),
    mesh=_mesh,
    scratch_types=[
        pltpu.VMEM((IB, CHUNK), jnp.int32),
        pltpu.VMEM((IB, CHUNK), jnp.int32),
        pltpu.VMEM((CHUNK, D), jnp.float32),
        pltpu.VMEM((CHUNK, D), jnp.float32),
        pltpu.SemaphoreType.DMA,
        pltpu.SemaphoreType.DMA,
        pltpu.VMEM_SHARED((NPAD, D), jnp.float32),
    ],
)
def _sc_agg1(table_hbm, src_hbm, dst_hbm, zero_hbm, out_hbm,
             src_v, dst_v, rows0_v, rows1_v, sem0, sem1, acc_sh):
    _edge_agg_body(NITER1, table_hbm, src_hbm, dst_hbm, zero_hbm, out_hbm,
                   src_v, dst_v, rows0_v, rows1_v, sem0, sem1, acc_sh,
                   split_by_core=False)


@functools.partial(
    pl.kernel,
    out_type=jax.ShapeDtypeStruct((NC, N, D), jnp.float32),
    mesh=_mesh,
    scratch_types=[
        pltpu.VMEM((IB, CHUNK), jnp.int32),
        pltpu.VMEM((IB, CHUNK), jnp.int32),
        pltpu.VMEM((CHUNK, D), jnp.float32),
        pltpu.VMEM((CHUNK, D), jnp.float32),
        pltpu.SemaphoreType.DMA,
        pltpu.SemaphoreType.DMA,
        pltpu.VMEM_SHARED((NPAD, D), jnp.float32),
    ],
)
def _sc_agg2(table_hbm, src_hbm, dst_hbm, zero_hbm, out_hbm,
             src_v, dst_v, rows0_v, rows1_v, sem0, sem1, acc_sh):
    _edge_agg_body(NITER2, table_hbm, src_hbm, dst_hbm, zero_hbm, out_hbm,
                   src_v, dst_v, rows0_v, rows1_v, sem0, sem1, acc_sh,
                   split_by_core=True)


BN = 2000


def _tc1_body(emb_ref, w1_ref, cnt_ref, xs_ref, dinv_ref):
    x1 = jnp.dot(emb_ref[...], w1_ref[...], preferred_element_type=jnp.float32)
    deg = 1.0 + cnt_ref[0, :, 0:1] + cnt_ref[1, :, 0:1]
    dinv = lax.rsqrt(deg)
    dinv_ref[...] = dinv
    xs = x1 * dinv
    xs_ref[0] = xs[:, :D]
    xs_ref[1] = xs[:, D:]


def _tc1(emb, W1, cnt):
    return pl.pallas_call(
        _tc1_body,
        grid=(N // BN,),
        in_specs=[
            pl.BlockSpec((BN, D), lambda i: (i, 0)),
            pl.BlockSpec((D, 2 * D), lambda i: (0, 0)),
            pl.BlockSpec((NC, BN, D), lambda i: (0, i, 0)),
        ],
        out_specs=[
            pl.BlockSpec((NC, BN, D), lambda i: (0, i, 0)),
            pl.BlockSpec((BN, 1), lambda i: (i, 0)),
        ],
        out_shape=[
            jax.ShapeDtypeStruct((NC, N, D), jnp.float32),
            jax.ShapeDtypeStruct((N, 1), jnp.float32),
        ],
    )(emb, W1, cnt)


def _tc2_body(agg_ref, xs_ref, dinv_ref, b1_ref, w2_ref, xs2_ref):
    dinv = dinv_ref[...]
    h = jnp.concatenate(
        [agg_ref[0] + xs_ref[0], agg_ref[1] + xs_ref[1]], axis=1)
    h = h * dinv + b1_ref[...]
    nrm = jnp.sqrt(jnp.sum(h * h, axis=1, keepdims=True))
    h = h / jnp.maximum(nrm, 1e-12)
    x2 = jnp.dot(h, w2_ref[...], preferred_element_type=jnp.float32)
    xs2_ref[...] = x2 * dinv


def _tc2(agg, xs, dinv, b1, W2):
    return pl.pallas_call(
        _tc2_body,
        grid=(N // BN,),
        in_specs=[
            pl.BlockSpec((NC, BN, D), lambda i: (0, i, 0)),
            pl.BlockSpec((NC, BN, D), lambda i: (0, i, 0)),
            pl.BlockSpec((BN, 1), lambda i: (i, 0)),
            pl.BlockSpec((1, 2 * D), lambda i: (0, 0)),
            pl.BlockSpec((2 * D, D), lambda i: (0, 0)),
        ],
        out_specs=pl.BlockSpec((BN, D), lambda i: (i, 0)),
        out_shape=jax.ShapeDtypeStruct((N, D), jnp.float32),
    )(agg, xs, dinv, b1, W2)


def _tc3_body(p_ref, xs2_ref, dinv_ref, b2_ref, out_ref):
    out_ref[...] = (dinv_ref[...] * (p_ref[0] + p_ref[1] + xs2_ref[...])
                    + b2_ref[...])


def _tc3(p, xs2, dinv, b2):
    return pl.pallas_call(
        _tc3_body,
        grid=(N // BN,),
        in_specs=[
            pl.BlockSpec((NC, BN, D), lambda i: (0, i, 0)),
            pl.BlockSpec((BN, D), lambda i: (i, 0)),
            pl.BlockSpec((BN, 1), lambda i: (i, 0)),
            pl.BlockSpec((1, D), lambda i: (0, 0)),
        ],
        out_specs=pl.BlockSpec((BN, D), lambda i: (i, 0)),
        out_shape=jax.ShapeDtypeStruct((N, D), jnp.float32),
    )(p, xs2, dinv, b2)


def kernel(edge_index, emb, W1, b1, W2, b2):
    src = edge_index[0].astype(jnp.int32)
    dst = edge_index[1].astype(jnp.int32)
    pad = E_PAD - E
    ar = jnp.arange(pad, dtype=jnp.int32)
    src_p = jnp.concatenate([src, (ar * 997) % N])
    dst_p = jnp.concatenate([dst, N + (ar % (NPAD - N))])

    src32 = src_p.reshape(NW, NITER2, CHUNK)
    dst32 = dst_p.reshape(NW, NITER2, CHUNK)
    src16 = src_p.reshape(NS, NITER1, CHUNK)
    src16_2 = jnp.stack([src16, src16 + N])
    dst16 = dst_p.reshape(NS, NITER1, CHUNK)

    ones128 = jnp.ones((CHUNK, D), jnp.float32)
    zeros128 = jnp.zeros((ZROWS, D), jnp.float32)

    cnt = _sc_degree(dst32, ones128, zeros128)
    xs, dinv = _tc1(emb, W1, cnt)
    agg = _sc_agg1(xs.reshape(NC * N, D), src16_2, dst16, zeros128)
    xs2 = _tc2(agg, xs, dinv, b1.reshape(1, 2 * D), W2)
    p = _sc_agg2(xs2, src32, dst32, zeros128)
    return _tc3(p, xs2, dinv, b2.reshape(1, D))
